# baseline (speedup 1.0000x reference)
import functools

import jax
import jax.numpy as jnp
from jax.experimental import pallas as pl

_G = 8


def _gcn_fused(a_ref, f_ref, w1_ref, b1_ref, w2_ref, b2_ref, out_ref, *, G, N):
    absa = jnp.abs(a_ref[...])
    d = jax.lax.rsqrt(jnp.sum(absa, axis=2, keepdims=True) + 1.0)

    w1 = w1_ref[...]
    w2 = w2_ref[...]
    b1 = b1_ref[...]
    b2 = b2_ref[...]

    s1_all = jnp.dot(f_ref[...], w1, preferred_element_type=jnp.float32)

    t1 = [d[g] * s1_all[N * g:N * g + N, :] for g in range(G)]
    p1 = [jnp.dot(absa[g], t1[g], preferred_element_type=jnp.float32) + t1[g]
          for g in range(G)]
    h1 = [jnp.maximum(d[g] * p1[g] + b1, 0.0) for g in range(G)]
    s2 = [jnp.dot(h1[g], w2, preferred_element_type=jnp.float32)
          for g in range(G)]
    t2 = [d[g] * s2[g] for g in range(G)]
    p2 = [jnp.dot(absa[g], t2[g], preferred_element_type=jnp.float32) + t2[g]
          for g in range(G)]
    for g in range(G):
        out_ref[N * g:N * g + N, :] = jnp.maximum(d[g] * p2[g] + b2, 0.0)


def kernel(a, f, W1, b1, W2, b2):
    B, N, _ = a.shape
    C = f.shape[2]
    H = W2.shape[1]
    G = _G
    out = pl.pallas_call(
        functools.partial(_gcn_fused, G=G, N=N),
        grid=(B // G,),
        in_specs=[
            pl.BlockSpec((G, N, N), lambda b: (b, 0, 0)),
            pl.BlockSpec((G * N, C), lambda b: (b, 0)),
            pl.BlockSpec((C, H), lambda b: (0, 0)),
            pl.BlockSpec((1, H), lambda b: (0, 0)),
            pl.BlockSpec((H, H), lambda b: (0, 0)),
            pl.BlockSpec((1, H), lambda b: (0, 0)),
        ],
        out_specs=pl.BlockSpec((G * N, H), lambda b: (b, 0)),
        out_shape=jax.ShapeDtypeStruct((B * N, H), jnp.float32),
    )(a, f.reshape(B * N, C), W1, b1.reshape(1, -1), W2, b2.reshape(1, -1))
    return out.reshape(B, N, H)


# speedup vs baseline: 1.7454x; 1.7454x over previous
import functools

import jax
import jax.numpy as jnp
from jax.experimental import pallas as pl

_G = 16


def _gcn_fused(a_ref, f_ref, w1_ref, b1_ref, w2_ref, b2_ref, out_ref, *, G):
    absa = jnp.abs(a_ref[...])
    d = jax.lax.rsqrt(jnp.sum(absa, axis=2, keepdims=True) + 1.0)
    absa_h = absa.astype(jnp.bfloat16)

    w1 = w1_ref[...].astype(jnp.bfloat16)
    w2 = w2_ref[...].astype(jnp.bfloat16)
    b1 = b1_ref[...]
    b2 = b2_ref[...]

    def mm(x, y):
        return jnp.dot(x, y, preferred_element_type=jnp.float32)

    s1 = [mm(f_ref[g].astype(jnp.bfloat16), w1) for g in range(G)]
    t1 = [(d[g] * s1[g]) for g in range(G)]
    p1 = [mm(absa_h[g], t1[g].astype(jnp.bfloat16)) + t1[g] for g in range(G)]
    h1 = [jnp.maximum(d[g] * p1[g] + b1, 0.0) for g in range(G)]
    s2 = [mm(h1[g].astype(jnp.bfloat16), w2) for g in range(G)]
    t2 = [(d[g] * s2[g]) for g in range(G)]
    p2 = [mm(absa_h[g], t2[g].astype(jnp.bfloat16)) + t2[g] for g in range(G)]
    for g in range(G):
        out_ref[g] = jnp.maximum(d[g] * p2[g] + b2, 0.0)


def kernel(a, f, W1, b1, W2, b2):
    B, N, _ = a.shape
    C = f.shape[2]
    H = W2.shape[1]
    G = _G
    return pl.pallas_call(
        functools.partial(_gcn_fused, G=G),
        grid=(B // G,),
        in_specs=[
            pl.BlockSpec((G, N, N), lambda b: (b, 0, 0)),
            pl.BlockSpec((G, N, C), lambda b: (b, 0, 0)),
            pl.BlockSpec((C, H), lambda b: (0, 0)),
            pl.BlockSpec((1, H), lambda b: (0, 0)),
            pl.BlockSpec((H, H), lambda b: (0, 0)),
            pl.BlockSpec((1, H), lambda b: (0, 0)),
        ],
        out_specs=pl.BlockSpec((G, N, H), lambda b: (b, 0, 0)),
        out_shape=jax.ShapeDtypeStruct((B, N, H), jnp.float32),
    )(a, f, W1, b1.reshape(1, -1), W2, b2.reshape(1, -1))
